# Initial kernel scaffold; baseline (speedup 1.0000x reference)
import jax
import jax.numpy as jnp
from jax.experimental import pallas as pl


def kernel(x, c0_Wm, c0_bm, c0_Wih, c0_bih, c0_Whh, c0_bhh, c1_Wm, c1_bm, c1_Wih, c1_bih, c1_Whh, c1_bhh, c2_Wm, c2_bm, c2_Wih, c2_bih, c2_Whh, c2_bhh, gate_W, gate_b, fc1_W, fc1_b, fc2_W, fc2_b, fc3_W, fc3_b, edge_index, etype):
    raise NotImplementedError("write your pallas kernel here")


# speedup vs baseline: 1.5744x; 1.5744x over previous
import functools

import jax
import jax.numpy as jnp
from jax import lax
from jax.experimental import pallas as pl
from jax.experimental.pallas import tpu as pltpu
from jax.experimental.pallas import tpu_sc as plsc

N = 10000
D = 128
NP = 10240
E = 160000
EP = 163840
NC = 2
NS = 16
NW = NC * NS
EW = EP // NW
CH = 128
NCHUNK = EW // CH
SR = NP // NS
BLK = 256
NB = NP // BLK


def _sc_scatter_body(m_hbm, src_hbm, dst_hbm, zeros_hbm, out_hbm,
                     src_v, dst_v, rows_v, acc, sem):
    cid = lax.axis_index("c")
    sid = lax.axis_index("s")
    wid = sid * NC + cid
    r0 = pl.multiple_of(sid * SR, 8)
    pltpu.sync_copy(zeros_hbm.at[pl.ds(r0, SR)], acc.at[pl.ds(r0, SR)])
    plsc.subcore_barrier()
    base = wid * EW
    for c in range(NCHUNK):
        off = pl.multiple_of(base + c * CH, 8)
        pltpu.sync_copy(src_hbm.at[pl.ds(off, CH)], src_v)
        pltpu.sync_copy(dst_hbm.at[pl.ds(off, CH)], dst_v)
        pltpu.async_copy(m_hbm.at[src_v], rows_v, sem).wait()
        pltpu.sync_copy(rows_v, acc.at[dst_v], add=True)
    plsc.subcore_barrier()
    o0 = pl.multiple_of(cid * NP + sid * SR, 8)
    pltpu.sync_copy(acc.at[pl.ds(r0, SR)], out_hbm.at[pl.ds(o0, SR)])


def _make_sc_scatter():
    mesh = plsc.VectorSubcoreMesh(core_axis_name="c", subcore_axis_name="s")
    return pl.kernel(
        _sc_scatter_body,
        out_type=jax.ShapeDtypeStruct((NC * NP, D), jnp.float32),
        mesh=mesh,
        scratch_types=[
            pltpu.VMEM((CH,), jnp.int32),
            pltpu.VMEM((CH,), jnp.int32),
            pltpu.VMEM((CH, D), jnp.float32),
            pltpu.VMEM_SHARED((NP, D), jnp.float32),
            pltpu.SemaphoreType.DMA,
        ],
    )


@functools.cache
def _get_sc_scatter():
    return _make_sc_scatter()


def _mm_body(x_ref, w_ref, b_ref, o_ref):
    o_ref[...] = (jnp.dot(x_ref[...], w_ref[...],
                          preferred_element_type=jnp.float32) + b_ref[0:1, :])


def _make_mm(interpret=False):
    return pl.pallas_call(
        _mm_body,
        grid=(NB,),
        in_specs=[pl.BlockSpec((BLK, D), lambda i: (i, 0)),
                  pl.BlockSpec((D, D), lambda i: (0, 0)),
                  pl.BlockSpec((8, D), lambda i: (0, 0))],
        out_specs=pl.BlockSpec((BLK, D), lambda i: (i, 0)),
        out_shape=jax.ShapeDtypeStruct((NP, D), jnp.float32),
        interpret=interpret,
    )


def _gru_math(a0, a1, h, wih, bih, whh, bhh):
    a = a0[...] + a1[...]
    gi = jnp.dot(a, wih[...], preferred_element_type=jnp.float32) + bih[0:1, :]
    gh = jnp.dot(h[...], whh[...], preferred_element_type=jnp.float32) + bhh[0:1, :]
    r = jax.nn.sigmoid(gi[:, :D] + gh[:, :D])
    z = jax.nn.sigmoid(gi[:, D:2 * D] + gh[:, D:2 * D])
    n = jnp.tanh(gi[:, 2 * D:] + r * gh[:, 2 * D:])
    return (1.0 - z) * n + z * h[...]


def _step_em_body(a0, a1, h, wih, bih, whh, bhh, wm, bm, ho, mo, *, relu):
    hn = _gru_math(a0, a1, h, wih, bih, whh, bhh)
    if relu:
        hn = jnp.maximum(hn, 0.0)
    ho[...] = hn
    mo[...] = (jnp.dot(hn, wm[...], preferred_element_type=jnp.float32)
               + bm[0:1, :])


def _step_last_body(a0, a1, h, wih, bih, whh, bhh, ho):
    ho[...] = _gru_math(a0, a1, h, wih, bih, whh, bhh)


_W_SPECS = [pl.BlockSpec((D, 3 * D), lambda i: (0, 0)),
            pl.BlockSpec((8, 3 * D), lambda i: (0, 0)),
            pl.BlockSpec((D, 3 * D), lambda i: (0, 0)),
            pl.BlockSpec((8, 3 * D), lambda i: (0, 0))]
_ROW = pl.BlockSpec((BLK, D), lambda i: (i, 0))
_ROW1 = pl.BlockSpec((BLK, D), lambda i: (i + NB, 0))


def _make_step_em(relu, interpret=False):
    return pl.pallas_call(
        functools.partial(_step_em_body, relu=relu),
        grid=(NB,),
        in_specs=[_ROW, _ROW1, _ROW] + _W_SPECS
                 + [pl.BlockSpec((D, D), lambda i: (0, 0)),
                    pl.BlockSpec((8, D), lambda i: (0, 0))],
        out_specs=[_ROW, _ROW],
        out_shape=[jax.ShapeDtypeStruct((NP, D), jnp.float32),
                   jax.ShapeDtypeStruct((NP, D), jnp.float32)],
        interpret=interpret,
    )


def _make_step_last(interpret=False):
    return pl.pallas_call(
        _step_last_body,
        grid=(NB,),
        in_specs=[_ROW, _ROW1, _ROW] + _W_SPECS,
        out_specs=_ROW,
        out_shape=jax.ShapeDtypeStruct((NP, D), jnp.float32),
        interpret=interpret,
    )


def _final_body(h1, h2, h3, gw, w1, b1, w2, b2, w3, b3, out):
    rows = lax.broadcasted_iota(jnp.int32, (NP, 1), 0)
    valid = rows < N
    g = (jnp.sum(h1[...] * gw[0:1, :], axis=1, keepdims=True)
         + jnp.sum(h2[...] * gw[1:2, :], axis=1, keepdims=True)
         + jnp.sum(h3[...] * gw[2:3, :], axis=1, keepdims=True))
    g = jnp.where(valid, g, -jnp.inf)
    e = jnp.where(valid, jnp.exp(g - jnp.max(g)), 0.0)
    s = jnp.sum(e)
    r1 = jnp.sum(h1[...] * e, axis=0, keepdims=True)
    r2 = jnp.sum(h2[...] * e, axis=0, keepdims=True)
    r3 = jnp.sum(h3[...] * e, axis=0, keepdims=True)
    fr = jnp.concatenate([r1, r2, r3], axis=1) / s
    o = jnp.maximum(jnp.dot(fr, w1[...],
                            preferred_element_type=jnp.float32) + b1[0:1, :], 0.0)
    o = jnp.maximum(jnp.dot(o, w2[...],
                            preferred_element_type=jnp.float32) + b2[0:1, :], 0.0)
    o = jnp.dot(o, w3[...], preferred_element_type=jnp.float32) + b3[0:1, :]
    out[...] = jnp.broadcast_to(jax.nn.sigmoid(o[0:1, 0:1]), (8, 128))


def _make_final(interpret=False):
    full = lambda *s: pl.BlockSpec(s, lambda: tuple(0 for _ in s))
    return pl.pallas_call(
        _final_body,
        in_specs=[full(NP, D), full(NP, D), full(NP, D),
                  full(8, D),
                  full(3 * D, 128), full(8, 128),
                  full(128, 128), full(8, 128),
                  full(128, 128), full(8, 128)],
        out_specs=full(8, 128),
        out_shape=jax.ShapeDtypeStruct((8, 128), jnp.float32),
        interpret=interpret,
    )


_mm = _make_mm()
_step_em = _make_step_em(False)
_step_em_relu = _make_step_em(True)
_step_last = _make_step_last()
_final = _make_final()


def _b8(b):
    return jnp.broadcast_to(b.reshape(1, -1), (8, b.shape[0]))


def _pad2(a, r, c):
    return jnp.pad(a, ((0, r - a.shape[0]), (0, c - a.shape[1])))


def kernel(x, c0_Wm, c0_bm, c0_Wih, c0_bih, c0_Whh, c0_bhh, c1_Wm, c1_bm,
           c1_Wih, c1_bih, c1_Whh, c1_bhh, c2_Wm, c2_bm, c2_Wih, c2_bih,
           c2_Whh, c2_bhh, gate_W, gate_b, fc1_W, fc1_b, fc2_W, fc2_b,
           fc3_W, fc3_b, edge_index, etype):
    del gate_b
    x_pad = jnp.pad(x, ((0, NP - N), (0, 0)))
    zeros = jnp.zeros((NP, D), jnp.float32)
    src = jnp.concatenate([edge_index[0],
                           jnp.zeros((EP - E,), jnp.int32)])
    dst = edge_index[1]
    dsts = []
    for et in (1, 2, 3):
        det = jnp.where(etype == et, dst, N).astype(jnp.int32)
        dsts.append(jnp.concatenate(
            [det, jnp.full((EP - E,), N, jnp.int32)]))

    convs = []
    for (Wm, bm, Wih, bih, Whh, bhh) in (
            (c0_Wm, c0_bm, c0_Wih, c0_bih, c0_Whh, c0_bhh),
            (c1_Wm, c1_bm, c1_Wih, c1_bih, c1_Whh, c1_bhh),
            (c2_Wm, c2_bm, c2_Wih, c2_bih, c2_Whh, c2_bhh)):
        convs.append((Wm.T, _b8(bm), Wih.T, _b8(bih), Whh.T, _b8(bhh)))

    sc_scatter = _get_sc_scatter()
    m0 = _mm(x_pad, convs[0][0], convs[0][1])
    subs = []
    for ei in range(3):
        h, m = x_pad, m0
        for ci in range(3):
            wm_t, bm8, wih_t, bih8, whh_t, bhh8 = convs[ci]
            for t in range(6):
                accf = sc_scatter(m, src, dsts[ei], zeros)
                if t < 5:
                    h, m = _step_em(accf, accf, h, wih_t, bih8, whh_t, bhh8,
                                    wm_t, bm8)
                elif ci < 2:
                    nwm_t, nbm8 = convs[ci + 1][0], convs[ci + 1][1]
                    h, m = _step_em_relu(accf, accf, h, wih_t, bih8,
                                         whh_t, bhh8, nwm_t, nbm8)
                else:
                    h = _step_last(accf, accf, h, wih_t, bih8, whh_t, bhh8)
        subs.append(h)

    gw = jnp.pad(gate_W.reshape(3, D), ((0, 5), (0, 0)))
    w1 = _pad2(fc1_W.T, 3 * D, 128)
    b1 = _b8(jnp.pad(fc1_b, (0, 28)))
    w2 = _pad2(fc2_W.T, 128, 128)
    b2 = _b8(jnp.pad(fc2_b, (0, 64)))
    w3 = _pad2(fc3_W.T, 128, 128)
    b3 = _b8(jnp.pad(fc3_b, (0, 127)))
    out = _final(subs[0], subs[1], subs[2], gw, w1, b1, w2, b2, w3, b3)
    return jnp.reshape(out[0, 0], (1,))


# speedup vs baseline: 4.9909x; 3.1700x over previous
import functools

import jax
import jax.numpy as jnp
from jax import lax
from jax.experimental import pallas as pl
from jax.experimental.pallas import tpu as pltpu
from jax.experimental.pallas import tpu_sc as plsc

N = 10000
D = 128
NP = 10240
E = 160000
EP = 163840
NC = 2
NS = 16
NW = NC * NS
CH = 128
HALF = NP // 2
AR = HALF + 256
ZR = AR // NS
OR = HALF // NS
SRNG = 320
SW = SRNG * D
CAP2 = 8192
ECH = 8192
NECH = EP // ECH
BLK = 256
NB = NP // BLK


def _compact_body(src_hbm, dst_hbm, et_hbm, pkq_hbm, cnt_hbm,
                  src_v, dst_v, et_v, pkb_v, cntb_v):
    cid = lax.axis_index("c")
    sid = lax.axis_index("s")
    tid = cid * NS + sid
    dhi = cid * HALF
    lo0 = sid * (2 * SRNG)

    def scan_chunk(c, ps):
        e0 = pl.multiple_of(c * ECH, 8)
        pltpu.sync_copy(src_hbm.at[pl.ds(e0, ECH)], src_v)
        pltpu.sync_copy(dst_hbm.at[pl.ds(e0, ECH)], dst_v)
        pltpu.sync_copy(et_hbm.at[pl.ds(e0, ECH)], et_v)

        def scan_vec(i, ps):
            o = i * 16
            s16 = src_v[pl.ds(o, 16)]
            d16 = dst_v[pl.ds(o, 16)]
            e16 = et_v[pl.ds(o, 16)]
            dh = jnp.logical_and(d16 >= dhi, d16 < dhi + HALF)
            out = []
            for b in range(6):
                et = b // 2 + 1
                lo = lo0 + (b % 2) * SRNG
                m = jnp.logical_and(
                    jnp.logical_and(e16 == et, dh),
                    jnp.logical_and(s16 >= lo, s16 < lo + SRNG))
                pk = (d16 - dhi) | ((s16 - lo) << 16)
                p = jnp.minimum(ps[b], CAP2 - 256)
                plsc.store_compressed(pkb_v.at[pl.ds(b * CAP2 + p, 16)],
                                      pk, mask=m)
                out.append(ps[b] + plsc.all_reduce_population_count(m)[0])
            return tuple(out)

        return lax.fori_loop(0, ECH // 16, scan_vec, ps)

    ps = lax.fori_loop(0, NECH, scan_chunk,
                       tuple(jnp.int32(0) for _ in range(6)))

    dummy = jnp.full((16,), HALF, jnp.int32)
    ones = jnp.ones((16,), jnp.bool_)
    for b in range(6):
        pc = jnp.minimum(ps[b], CAP2 - 256)
        for j in range(CH // 16):
            plsc.store_compressed(pkb_v.at[pl.ds(b * CAP2 + pc + j * 16, 16)],
                                  dummy, mask=ones)
        cntb_v[pl.ds(b * 16, 16)] = jnp.full(
            (16,), (pc + (CH - 1)) // CH, jnp.int32)

    for b in range(6):
        et, rng = b // 2, b % 2
        reg = et * (2 * NW) + tid * 2 + rng
        q0 = pl.multiple_of(reg * CAP2, 8)
        pltpu.sync_copy(pkb_v.at[pl.ds(b * CAP2, CAP2)],
                        pkq_hbm.at[pl.ds(q0, CAP2)])
    for et in range(3):
        c0 = pl.multiple_of((et * (2 * NW) + tid * 2) * 16, 8)
        pltpu.sync_copy(cntb_v.at[pl.ds((et * 2) * 16, 32)],
                        cnt_hbm.at[pl.ds(c0, 32)])


def _make_compact():
    mesh = plsc.VectorSubcoreMesh(core_axis_name="c", subcore_axis_name="s")
    return pl.kernel(
        _compact_body,
        out_type=(jax.ShapeDtypeStruct((3 * 2 * NW * CAP2,), jnp.int32),
                  jax.ShapeDtypeStruct((3 * 2 * NW * 16,), jnp.int32)),
        mesh=mesh,
        compiler_params=pltpu.CompilerParams(needs_layout_passes=False),
        scratch_types=[
            pltpu.VMEM((ECH,), jnp.int32),
            pltpu.VMEM((ECH,), jnp.int32),
            pltpu.VMEM((ECH,), jnp.int32),
            pltpu.VMEM((6 * CAP2,), jnp.int32),
            pltpu.VMEM((6 * 16,), jnp.int32),
        ],
    )


def _sc_scatter_body(m_hbm, pkq_hbm, cnt_hbm, zeros_hbm, out_hbm,
                     ml_v, pk_v, chunk_v, drow_v, cnt_v, acc, sem):
    cid = lax.axis_index("c")
    sid = lax.axis_index("s")
    tid = cid * NS + sid
    z0 = pl.multiple_of(sid * ZR, 8)
    pltpu.sync_copy(zeros_hbm.at[pl.ds(z0, ZR)], acc.at[pl.ds(z0, ZR)])
    plsc.subcore_barrier()

    for rng in range(2):
        lo = sid * (2 * SRNG) + rng * SRNG
        pltpu.sync_copy(m_hbm.at[pl.ds(lo * D, SW)], ml_v)
        reg = tid * 2 + rng
        q0 = pl.multiple_of(reg * CAP2, 8)
        pltpu.sync_copy(pkq_hbm.at[pl.ds(q0, CAP2)], pk_v)
        c0 = pl.multiple_of(reg * 16, 8)
        pltpu.sync_copy(cnt_hbm.at[pl.ds(c0, 16)], cnt_v)
        nch = cnt_v[pl.ds(0, 16)][0]

        def chunk_body(k, carry):
            for g in range(CH // 16):
                pk16 = pk_v[pl.ds(k * CH + g * 16, 16)]
                d16 = pk16 & 0xFFFF
                s16 = pk16 >> 16
                drow_v[pl.ds(g * 16, 16)] = d16
                base = s16 * D
                for l in range(16):
                    b0 = base[l]
                    for j in range(D // 16):
                        chunk_v[g * 16 + l, pl.ds(j * 16, 16)] = (
                            ml_v[pl.ds(b0 + j * 16, 16)])
            pltpu.sync_copy(chunk_v, acc.at[drow_v], add=True)
            return carry

        lax.fori_loop(0, nch, chunk_body, jnp.int32(0))

    plsc.subcore_barrier()
    a0 = pl.multiple_of(sid * OR, 8)
    o0 = pl.multiple_of(cid * HALF + sid * OR, 8)
    pltpu.sync_copy(acc.at[pl.ds(a0, OR)], out_hbm.at[pl.ds(o0, OR)])


def _make_sc_scatter():
    mesh = plsc.VectorSubcoreMesh(core_axis_name="c", subcore_axis_name="s")
    return pl.kernel(
        _sc_scatter_body,
        out_type=jax.ShapeDtypeStruct((NP, D), jnp.float32),
        mesh=mesh,
        compiler_params=pltpu.CompilerParams(needs_layout_passes=False),
        scratch_types=[
            pltpu.VMEM((SW,), jnp.float32),
            pltpu.VMEM((CAP2,), jnp.int32),
            pltpu.VMEM((CH, D), jnp.float32),
            pltpu.VMEM((CH,), jnp.int32),
            pltpu.VMEM((16,), jnp.int32),
            pltpu.VMEM_SHARED((AR, D), jnp.float32),
            pltpu.SemaphoreType.DMA,
        ],
    )


@functools.cache
def _get_sc_kernels():
    return _make_compact(), _make_sc_scatter()


def _mm_body(x_ref, w_ref, b_ref, o_ref):
    o_ref[...] = (jnp.dot(x_ref[...], w_ref[...],
                          preferred_element_type=jnp.float32) + b_ref[0:1, :])


def _make_mm(interpret=False):
    return pl.pallas_call(
        _mm_body,
        grid=(NB,),
        in_specs=[pl.BlockSpec((BLK, D), lambda i: (i, 0)),
                  pl.BlockSpec((D, D), lambda i: (0, 0)),
                  pl.BlockSpec((8, D), lambda i: (0, 0))],
        out_specs=pl.BlockSpec((BLK, D), lambda i: (i, 0)),
        out_shape=jax.ShapeDtypeStruct((NP, D), jnp.float32),
        interpret=interpret,
    )


def _gru_math(a0, h, wih, bih, whh, bhh):
    gi = jnp.dot(a0[...], wih[...],
                 preferred_element_type=jnp.float32) + bih[0:1, :]
    gh = jnp.dot(h[...], whh[...],
                 preferred_element_type=jnp.float32) + bhh[0:1, :]
    r = jax.nn.sigmoid(gi[:, :D] + gh[:, :D])
    z = jax.nn.sigmoid(gi[:, D:2 * D] + gh[:, D:2 * D])
    n = jnp.tanh(gi[:, 2 * D:] + r * gh[:, 2 * D:])
    return (1.0 - z) * n + z * h[...]


def _step_em_body(a0, h, wih, bih, whh, bhh, wm, bm, ho, mo, *, relu):
    hn = _gru_math(a0, h, wih, bih, whh, bhh)
    if relu:
        hn = jnp.maximum(hn, 0.0)
    ho[...] = hn
    mo[...] = (jnp.dot(hn, wm[...], preferred_element_type=jnp.float32)
               + bm[0:1, :])


def _step_last_body(a0, h, wih, bih, whh, bhh, ho):
    ho[...] = _gru_math(a0, h, wih, bih, whh, bhh)


_W_SPECS = [pl.BlockSpec((D, 3 * D), lambda i: (0, 0)),
            pl.BlockSpec((8, 3 * D), lambda i: (0, 0)),
            pl.BlockSpec((D, 3 * D), lambda i: (0, 0)),
            pl.BlockSpec((8, 3 * D), lambda i: (0, 0))]
_ROW = pl.BlockSpec((BLK, D), lambda i: (i, 0))


def _make_step_em(relu, interpret=False):
    return pl.pallas_call(
        functools.partial(_step_em_body, relu=relu),
        grid=(NB,),
        in_specs=[_ROW, _ROW] + _W_SPECS
                 + [pl.BlockSpec((D, D), lambda i: (0, 0)),
                    pl.BlockSpec((8, D), lambda i: (0, 0))],
        out_specs=[_ROW, _ROW],
        out_shape=[jax.ShapeDtypeStruct((NP, D), jnp.float32),
                   jax.ShapeDtypeStruct((NP, D), jnp.float32)],
        interpret=interpret,
    )


def _make_step_last(interpret=False):
    return pl.pallas_call(
        _step_last_body,
        grid=(NB,),
        in_specs=[_ROW, _ROW] + _W_SPECS,
        out_specs=_ROW,
        out_shape=jax.ShapeDtypeStruct((NP, D), jnp.float32),
        interpret=interpret,
    )


def _final_body(h1, h2, h3, gw, w1, b1, w2, b2, w3, b3, out):
    rows = lax.broadcasted_iota(jnp.int32, (NP, 1), 0)
    valid = rows < N
    g = (jnp.sum(h1[...] * gw[0:1, :], axis=1, keepdims=True)
         + jnp.sum(h2[...] * gw[1:2, :], axis=1, keepdims=True)
         + jnp.sum(h3[...] * gw[2:3, :], axis=1, keepdims=True))
    g = jnp.where(valid, g, -jnp.inf)
    e = jnp.where(valid, jnp.exp(g - jnp.max(g)), 0.0)
    s = jnp.sum(e)
    r1 = jnp.sum(h1[...] * e, axis=0, keepdims=True)
    r2 = jnp.sum(h2[...] * e, axis=0, keepdims=True)
    r3 = jnp.sum(h3[...] * e, axis=0, keepdims=True)
    fr = jnp.concatenate([r1, r2, r3], axis=1) / s
    o = jnp.maximum(jnp.dot(fr, w1[...],
                            preferred_element_type=jnp.float32) + b1[0:1, :], 0.0)
    o = jnp.maximum(jnp.dot(o, w2[...],
                            preferred_element_type=jnp.float32) + b2[0:1, :], 0.0)
    o = jnp.dot(o, w3[...], preferred_element_type=jnp.float32) + b3[0:1, :]
    out[...] = jnp.broadcast_to(jax.nn.sigmoid(o[0:1, 0:1]), (8, 128))


def _make_final(interpret=False):
    full = lambda *s: pl.BlockSpec(s, lambda: tuple(0 for _ in s))
    return pl.pallas_call(
        _final_body,
        in_specs=[full(NP, D), full(NP, D), full(NP, D),
                  full(8, D),
                  full(3 * D, 128), full(8, 128),
                  full(128, 128), full(8, 128),
                  full(128, 128), full(8, 128)],
        out_specs=full(8, 128),
        out_shape=jax.ShapeDtypeStruct((8, 128), jnp.float32),
        interpret=interpret,
    )


_mm = _make_mm()
_step_em = _make_step_em(False)
_step_em_relu = _make_step_em(True)
_step_last = _make_step_last()
_final = _make_final()


def _b8(b):
    return jnp.broadcast_to(b.reshape(1, -1), (8, b.shape[0]))


def _pad2(a, r, c):
    return jnp.pad(a, ((0, r - a.shape[0]), (0, c - a.shape[1])))


def kernel(x, c0_Wm, c0_bm, c0_Wih, c0_bih, c0_Whh, c0_bhh, c1_Wm, c1_bm,
           c1_Wih, c1_bih, c1_Whh, c1_bhh, c2_Wm, c2_bm, c2_Wih, c2_bih,
           c2_Whh, c2_bhh, gate_W, gate_b, fc1_W, fc1_b, fc2_W, fc2_b,
           fc3_W, fc3_b, edge_index, etype):
    del gate_b
    x_pad = jnp.pad(x, ((0, NP - N), (0, 0)))
    zeros = jnp.zeros((AR, D), jnp.float32)
    src = jnp.concatenate([edge_index[0],
                           jnp.zeros((EP - E,), jnp.int32)])
    dst = jnp.concatenate([edge_index[1],
                           jnp.zeros((EP - E,), jnp.int32)])
    etp = jnp.concatenate([etype, jnp.zeros((EP - E,), jnp.int32)])

    compact, sc_scatter = _get_sc_kernels()
    pkq, cnts = compact(src, dst, etp)
    reg_sz = 2 * NW * CAP2
    et_pkq = [pkq[ei * reg_sz:(ei + 1) * reg_sz] for ei in range(3)]
    et_cnt = [cnts[ei * 2 * NW * 16:(ei + 1) * 2 * NW * 16] for ei in range(3)]

    convs = []
    for (Wm, bm, Wih, bih, Whh, bhh) in (
            (c0_Wm, c0_bm, c0_Wih, c0_bih, c0_Whh, c0_bhh),
            (c1_Wm, c1_bm, c1_Wih, c1_bih, c1_Whh, c1_bhh),
            (c2_Wm, c2_bm, c2_Wih, c2_bih, c2_Whh, c2_bhh)):
        convs.append((Wm.T, _b8(bm), Wih.T, _b8(bih), Whh.T, _b8(bhh)))

    m0 = _mm(x_pad, convs[0][0], convs[0][1])
    subs = []
    for ei in range(3):
        h, m = x_pad, m0
        for ci in range(3):
            wm_t, bm8, wih_t, bih8, whh_t, bhh8 = convs[ci]
            for t in range(6):
                acc = sc_scatter(m.reshape(NP * D), et_pkq[ei], et_cnt[ei],
                                 zeros)
                if t < 5:
                    h, m = _step_em(acc, h, wih_t, bih8, whh_t, bhh8,
                                    wm_t, bm8)
                elif ci < 2:
                    nwm_t, nbm8 = convs[ci + 1][0], convs[ci + 1][1]
                    h, m = _step_em_relu(acc, h, wih_t, bih8,
                                         whh_t, bhh8, nwm_t, nbm8)
                else:
                    h = _step_last(acc, h, wih_t, bih8, whh_t, bhh8)
        subs.append(h)

    gw = jnp.pad(gate_W.reshape(3, D), ((0, 5), (0, 0)))
    w1 = _pad2(fc1_W.T, 3 * D, 128)
    b1 = _b8(jnp.pad(fc1_b, (0, 28)))
    w2 = _pad2(fc2_W.T, 128, 128)
    b2 = _b8(jnp.pad(fc2_b, (0, 64)))
    w3 = _pad2(fc3_W.T, 128, 128)
    b3 = _b8(jnp.pad(fc3_b, (0, 127)))
    out = _final(subs[0], subs[1], subs[2], gw, w1, b1, w2, b2, w3, b3)
    return jnp.reshape(out[0, 0], (1,))


# speedup vs baseline: 5.1455x; 1.0310x over previous
import functools

import jax
import jax.numpy as jnp
from jax import lax
from jax.experimental import pallas as pl
from jax.experimental.pallas import tpu as pltpu
from jax.experimental.pallas import tpu_sc as plsc

N = 10000
D = 128
NP = 10240
E = 160000
EP = 163840
NC = 2
NS = 16
NW = NC * NS
CH = 128
HALF = NP // 2
AR = HALF + 256
ZR = AR // NS
OR = HALF // NS
SRNG = 320
SW = SRNG * D
CAP2 = 8192
ECH = 8192
NECH = EP // ECH
BLK = 256
NB = NP // BLK


def _compact_body(src_hbm, dst_hbm, et_hbm, pkq_hbm, cnt_hbm,
                  src_v, dst_v, et_v, pkb_v, cntb_v):
    cid = lax.axis_index("c")
    sid = lax.axis_index("s")
    tid = cid * NS + sid
    dhi = cid * HALF
    lo0 = sid * (2 * SRNG)

    def scan_chunk(c, ps):
        e0 = pl.multiple_of(c * ECH, 8)
        pltpu.sync_copy(src_hbm.at[pl.ds(e0, ECH)], src_v)
        pltpu.sync_copy(dst_hbm.at[pl.ds(e0, ECH)], dst_v)
        pltpu.sync_copy(et_hbm.at[pl.ds(e0, ECH)], et_v)

        def scan_vec(i, ps):
            o = i * 16
            s16 = src_v[pl.ds(o, 16)]
            d16 = dst_v[pl.ds(o, 16)]
            e16 = et_v[pl.ds(o, 16)]
            dh = jnp.logical_and(d16 >= dhi, d16 < dhi + HALF)
            out = []
            for b in range(6):
                et = b // 2 + 1
                lo = lo0 + (b % 2) * SRNG
                m = jnp.logical_and(
                    jnp.logical_and(e16 == et, dh),
                    jnp.logical_and(s16 >= lo, s16 < lo + SRNG))
                pk = (d16 - dhi) | ((s16 - lo) << 16)
                p = jnp.minimum(ps[b], CAP2 - 256)
                plsc.store_compressed(pkb_v.at[pl.ds(b * CAP2 + p, 16)],
                                      pk, mask=m)
                out.append(ps[b] + plsc.all_reduce_population_count(m)[0])
            return tuple(out)

        return lax.fori_loop(0, ECH // 16, scan_vec, ps)

    ps = lax.fori_loop(0, NECH, scan_chunk,
                       tuple(jnp.int32(0) for _ in range(6)))

    dummy = jnp.full((16,), HALF, jnp.int32)
    ones = jnp.ones((16,), jnp.bool_)
    for b in range(6):
        pc = jnp.minimum(ps[b], CAP2 - 256)
        for j in range(CH // 16):
            plsc.store_compressed(pkb_v.at[pl.ds(b * CAP2 + pc + j * 16, 16)],
                                  dummy, mask=ones)
        cntb_v[pl.ds(b * 16, 16)] = jnp.full(
            (16,), (pc + (CH - 1)) // CH, jnp.int32)

    for b in range(6):
        et, rng = b // 2, b % 2
        reg = et * (2 * NW) + tid * 2 + rng
        q0 = pl.multiple_of(reg * CAP2, 8)
        pltpu.sync_copy(pkb_v.at[pl.ds(b * CAP2, CAP2)],
                        pkq_hbm.at[pl.ds(q0, CAP2)])
    for et in range(3):
        c0 = pl.multiple_of((et * (2 * NW) + tid * 2) * 16, 8)
        pltpu.sync_copy(cntb_v.at[pl.ds((et * 2) * 16, 32)],
                        cnt_hbm.at[pl.ds(c0, 32)])


def _make_compact():
    mesh = plsc.VectorSubcoreMesh(core_axis_name="c", subcore_axis_name="s")
    return pl.kernel(
        _compact_body,
        out_type=(jax.ShapeDtypeStruct((3 * 2 * NW * CAP2,), jnp.int32),
                  jax.ShapeDtypeStruct((3 * 2 * NW * 16,), jnp.int32)),
        mesh=mesh,
        compiler_params=pltpu.CompilerParams(needs_layout_passes=False),
        scratch_types=[
            pltpu.VMEM((ECH,), jnp.int32),
            pltpu.VMEM((ECH,), jnp.int32),
            pltpu.VMEM((ECH,), jnp.int32),
            pltpu.VMEM((6 * CAP2,), jnp.int32),
            pltpu.VMEM((6 * 16,), jnp.int32),
        ],
    )


def _sc_scatter_body(m_hbm, pkq_hbm, cnt_hbm, zeros_hbm, out_hbm,
                     ml_v, pk_v, chunk01, drow01, sbase_v, cnt_v, acc,
                     sem):
    cid = lax.axis_index("c")
    sid = lax.axis_index("s")
    tid = cid * NS + sid
    z0 = pl.multiple_of(sid * ZR, 8)
    pltpu.sync_copy(zeros_hbm.at[pl.ds(z0, ZR)], acc.at[pl.ds(z0, ZR)])
    plsc.subcore_barrier()

    for rng in range(2):
        lo = sid * (2 * SRNG) + rng * SRNG
        pltpu.sync_copy(m_hbm.at[pl.ds(lo * D, SW)], ml_v)
        reg = tid * 2 + rng
        q0 = pl.multiple_of(reg * CAP2, 8)
        pltpu.sync_copy(pkq_hbm.at[pl.ds(q0, CAP2)], pk_v)
        c0 = pl.multiple_of(reg * 16, 8)
        pltpu.sync_copy(cnt_hbm.at[pl.ds(c0, 16)], cnt_v)
        nch = cnt_v[pl.ds(0, 16)][0]

        iota16 = lax.broadcasted_iota(jnp.int32, (16,), 0)

        def mat(k, ob):
            for g in range(CH // 16):
                pk16 = pk_v[pl.ds(k * CH + g * 16, 16)]
                drow01[pl.ds(ob + g * 16, 16)] = pk16 & 0xFFFF
                sbase_v[pl.ds(g * 16, 16)] = (pk16 >> 16) * D

            def lane(l, c):
                b0 = sbase_v[pl.ds(l, 16)][0]
                r16 = jnp.full((16,), ob + l, jnp.int32)
                for j in range(D // 16):
                    plsc.store_scatter(chunk01, [r16, iota16 + (j * 16)],
                                       ml_v[pl.ds(b0 + j * 16, 16)])
                return c

            lax.fori_loop(0, CH, lane, jnp.int32(0))

        @pl.when(nch > 0)
        def _():
            mat(jnp.int32(0), jnp.int32(0))

        def chunk_body(k, carry):
            ob = pl.multiple_of((k & 1) * CH, 8)
            src = chunk01.at[pl.ds(ob, CH)]
            idx = drow01.at[pl.ds(ob, CH)]
            pltpu.async_copy(src, acc.at[idx], sem, add=True)

            @pl.when(k + 1 < nch)
            def _():
                mat(k + 1, pl.multiple_of(((k + 1) & 1) * CH, 8))

            pltpu.make_async_copy(src, acc.at[idx], sem).wait()
            return carry

        lax.fori_loop(0, nch, chunk_body, jnp.int32(0))

    plsc.subcore_barrier()
    a0 = pl.multiple_of(sid * OR, 8)
    o0 = pl.multiple_of(cid * HALF + sid * OR, 8)
    pltpu.sync_copy(acc.at[pl.ds(a0, OR)], out_hbm.at[pl.ds(o0, OR)])


def _make_sc_scatter():
    mesh = plsc.VectorSubcoreMesh(core_axis_name="c", subcore_axis_name="s")
    return pl.kernel(
        _sc_scatter_body,
        out_type=jax.ShapeDtypeStruct((NP, D), jnp.float32),
        mesh=mesh,
        compiler_params=pltpu.CompilerParams(needs_layout_passes=False),
        scratch_types=[
            pltpu.VMEM((SW,), jnp.float32),
            pltpu.VMEM((CAP2,), jnp.int32),
            pltpu.VMEM((2 * CH, D), jnp.float32),
            pltpu.VMEM((2 * CH,), jnp.int32),
            pltpu.VMEM((CH + 16,), jnp.int32),
            pltpu.VMEM((16,), jnp.int32),
            pltpu.VMEM_SHARED((AR, D), jnp.float32),
            pltpu.SemaphoreType.DMA,
        ],
    )


@functools.cache
def _get_sc_kernels():
    return _make_compact(), _make_sc_scatter()


def _mm_body(x_ref, w_ref, b_ref, o_ref):
    o_ref[...] = (jnp.dot(x_ref[...], w_ref[...],
                          preferred_element_type=jnp.float32) + b_ref[0:1, :])


def _make_mm(interpret=False):
    return pl.pallas_call(
        _mm_body,
        grid=(NB,),
        in_specs=[pl.BlockSpec((BLK, D), lambda i: (i, 0)),
                  pl.BlockSpec((D, D), lambda i: (0, 0)),
                  pl.BlockSpec((8, D), lambda i: (0, 0))],
        out_specs=pl.BlockSpec((BLK, D), lambda i: (i, 0)),
        out_shape=jax.ShapeDtypeStruct((NP, D), jnp.float32),
        interpret=interpret,
    )


def _gru_math(a0, h, wih, bih, whh, bhh):
    gi = jnp.dot(a0[...], wih[...],
                 preferred_element_type=jnp.float32) + bih[0:1, :]
    gh = jnp.dot(h[...], whh[...],
                 preferred_element_type=jnp.float32) + bhh[0:1, :]
    r = jax.nn.sigmoid(gi[:, :D] + gh[:, :D])
    z = jax.nn.sigmoid(gi[:, D:2 * D] + gh[:, D:2 * D])
    n = jnp.tanh(gi[:, 2 * D:] + r * gh[:, 2 * D:])
    return (1.0 - z) * n + z * h[...]


def _step_em_body(a0, h, wih, bih, whh, bhh, wm, bm, ho, mo, *, relu):
    hn = _gru_math(a0, h, wih, bih, whh, bhh)
    if relu:
        hn = jnp.maximum(hn, 0.0)
    ho[...] = hn
    mo[...] = (jnp.dot(hn, wm[...], preferred_element_type=jnp.float32)
               + bm[0:1, :])


def _step_last_body(a0, h, wih, bih, whh, bhh, ho):
    ho[...] = _gru_math(a0, h, wih, bih, whh, bhh)


_W_SPECS = [pl.BlockSpec((D, 3 * D), lambda i: (0, 0)),
            pl.BlockSpec((8, 3 * D), lambda i: (0, 0)),
            pl.BlockSpec((D, 3 * D), lambda i: (0, 0)),
            pl.BlockSpec((8, 3 * D), lambda i: (0, 0))]
_ROW = pl.BlockSpec((BLK, D), lambda i: (i, 0))


def _make_step_em(relu, interpret=False):
    return pl.pallas_call(
        functools.partial(_step_em_body, relu=relu),
        grid=(NB,),
        in_specs=[_ROW, _ROW] + _W_SPECS
                 + [pl.BlockSpec((D, D), lambda i: (0, 0)),
                    pl.BlockSpec((8, D), lambda i: (0, 0))],
        out_specs=[_ROW, _ROW],
        out_shape=[jax.ShapeDtypeStruct((NP, D), jnp.float32),
                   jax.ShapeDtypeStruct((NP, D), jnp.float32)],
        interpret=interpret,
    )


def _make_step_last(interpret=False):
    return pl.pallas_call(
        _step_last_body,
        grid=(NB,),
        in_specs=[_ROW, _ROW] + _W_SPECS,
        out_specs=_ROW,
        out_shape=jax.ShapeDtypeStruct((NP, D), jnp.float32),
        interpret=interpret,
    )


def _final_body(h1, h2, h3, gw, w1, b1, w2, b2, w3, b3, out):
    rows = lax.broadcasted_iota(jnp.int32, (NP, 1), 0)
    valid = rows < N
    g = (jnp.sum(h1[...] * gw[0:1, :], axis=1, keepdims=True)
         + jnp.sum(h2[...] * gw[1:2, :], axis=1, keepdims=True)
         + jnp.sum(h3[...] * gw[2:3, :], axis=1, keepdims=True))
    g = jnp.where(valid, g, -jnp.inf)
    e = jnp.where(valid, jnp.exp(g - jnp.max(g)), 0.0)
    s = jnp.sum(e)
    r1 = jnp.sum(h1[...] * e, axis=0, keepdims=True)
    r2 = jnp.sum(h2[...] * e, axis=0, keepdims=True)
    r3 = jnp.sum(h3[...] * e, axis=0, keepdims=True)
    fr = jnp.concatenate([r1, r2, r3], axis=1) / s
    o = jnp.maximum(jnp.dot(fr, w1[...],
                            preferred_element_type=jnp.float32) + b1[0:1, :], 0.0)
    o = jnp.maximum(jnp.dot(o, w2[...],
                            preferred_element_type=jnp.float32) + b2[0:1, :], 0.0)
    o = jnp.dot(o, w3[...], preferred_element_type=jnp.float32) + b3[0:1, :]
    out[...] = jnp.broadcast_to(jax.nn.sigmoid(o[0:1, 0:1]), (8, 128))


def _make_final(interpret=False):
    full = lambda *s: pl.BlockSpec(s, lambda: tuple(0 for _ in s))
    return pl.pallas_call(
        _final_body,
        in_specs=[full(NP, D), full(NP, D), full(NP, D),
                  full(8, D),
                  full(3 * D, 128), full(8, 128),
                  full(128, 128), full(8, 128),
                  full(128, 128), full(8, 128)],
        out_specs=full(8, 128),
        out_shape=jax.ShapeDtypeStruct((8, 128), jnp.float32),
        interpret=interpret,
    )


_mm = _make_mm()
_step_em = _make_step_em(False)
_step_em_relu = _make_step_em(True)
_step_last = _make_step_last()
_final = _make_final()


def _b8(b):
    return jnp.broadcast_to(b.reshape(1, -1), (8, b.shape[0]))


def _pad2(a, r, c):
    return jnp.pad(a, ((0, r - a.shape[0]), (0, c - a.shape[1])))


def kernel(x, c0_Wm, c0_bm, c0_Wih, c0_bih, c0_Whh, c0_bhh, c1_Wm, c1_bm,
           c1_Wih, c1_bih, c1_Whh, c1_bhh, c2_Wm, c2_bm, c2_Wih, c2_bih,
           c2_Whh, c2_bhh, gate_W, gate_b, fc1_W, fc1_b, fc2_W, fc2_b,
           fc3_W, fc3_b, edge_index, etype):
    del gate_b
    x_pad = jnp.pad(x, ((0, NP - N), (0, 0)))
    zeros = jnp.zeros((AR, D), jnp.float32)
    src = jnp.concatenate([edge_index[0],
                           jnp.zeros((EP - E,), jnp.int32)])
    dst = jnp.concatenate([edge_index[1],
                           jnp.zeros((EP - E,), jnp.int32)])
    etp = jnp.concatenate([etype, jnp.zeros((EP - E,), jnp.int32)])

    compact, sc_scatter = _get_sc_kernels()
    pkq, cnts = compact(src, dst, etp)
    reg_sz = 2 * NW * CAP2
    et_pkq = [pkq[ei * reg_sz:(ei + 1) * reg_sz] for ei in range(3)]
    et_cnt = [cnts[ei * 2 * NW * 16:(ei + 1) * 2 * NW * 16] for ei in range(3)]

    convs = []
    for (Wm, bm, Wih, bih, Whh, bhh) in (
            (c0_Wm, c0_bm, c0_Wih, c0_bih, c0_Whh, c0_bhh),
            (c1_Wm, c1_bm, c1_Wih, c1_bih, c1_Whh, c1_bhh),
            (c2_Wm, c2_bm, c2_Wih, c2_bih, c2_Whh, c2_bhh)):
        convs.append((Wm.T, _b8(bm), Wih.T, _b8(bih), Whh.T, _b8(bhh)))

    m0 = _mm(x_pad, convs[0][0], convs[0][1])
    subs = []
    for ei in range(3):
        h, m = x_pad, m0
        for ci in range(3):
            wm_t, bm8, wih_t, bih8, whh_t, bhh8 = convs[ci]
            for t in range(6):
                acc = sc_scatter(m.reshape(NP * D), et_pkq[ei], et_cnt[ei],
                                 zeros)
                if t < 5:
                    h, m = _step_em(acc, h, wih_t, bih8, whh_t, bhh8,
                                    wm_t, bm8)
                elif ci < 2:
                    nwm_t, nbm8 = convs[ci + 1][0], convs[ci + 1][1]
                    h, m = _step_em_relu(acc, h, wih_t, bih8,
                                         whh_t, bhh8, nwm_t, nbm8)
                else:
                    h = _step_last(acc, h, wih_t, bih8, whh_t, bhh8)
        subs.append(h)

    gw = jnp.pad(gate_W.reshape(3, D), ((0, 5), (0, 0)))
    w1 = _pad2(fc1_W.T, 3 * D, 128)
    b1 = _b8(jnp.pad(fc1_b, (0, 28)))
    w2 = _pad2(fc2_W.T, 128, 128)
    b2 = _b8(jnp.pad(fc2_b, (0, 64)))
    w3 = _pad2(fc3_W.T, 128, 128)
    b3 = _b8(jnp.pad(fc3_b, (0, 127)))
    out = _final(subs[0], subs[1], subs[2], gw, w1, b1, w2, b2, w3, b3)
    return jnp.reshape(out[0, 0], (1,))
